# baseline (device time: 42775 ns/iter reference)
import jax
import jax.numpy as jnp
from jax import lax
from jax.experimental import pallas as pl
from jax.experimental.pallas import tpu as pltpu

N_DEV = 8
M = 768
N = 768
M_PER = M // N_DEV
N_HOPS = N_DEV - 1


def kernel(A, B):
    k_per = A.shape[1]

    def body(a_ref, b_ref, out_ref, part_ref, sendbuf, recvbuf,
             send_sems, recv_sems):
        p = lax.axis_index("i")
        left = jnp.mod(p + N_DEV - 1, N_DEV)
        right = jnp.mod(p + 1, N_DEV)

        barrier_sem = pltpu.get_barrier_semaphore()
        for nbr in [left, right]:
            pl.semaphore_signal(
                barrier_sem, inc=1,
                device_id=(nbr,), device_id_type=pl.DeviceIdType.MESH,
            )
        pl.semaphore_wait(barrier_sem, 2)

        part_ref[...] = jnp.dot(
            a_ref[...].astype(jnp.bfloat16),
            b_ref[...].astype(jnp.bfloat16),
            preferred_element_type=jnp.float32,
        )

        sendbuf[0] = part_ref[pl.ds((p + N_DEV - 1) % N_DEV * M_PER, M_PER), :]
        for s in range(N_HOPS):
            rdma = pltpu.make_async_remote_copy(
                src_ref=sendbuf.at[s],
                dst_ref=recvbuf.at[s],
                send_sem=send_sems.at[s],
                recv_sem=recv_sems.at[s],
                device_id=(right,),
                device_id_type=pl.DeviceIdType.MESH,
            )
            rdma.start()
            rdma.wait()

            c = (p + 2 * N_DEV - s - 2) % N_DEV
            acc = recvbuf[s] + part_ref[pl.ds(c * M_PER, M_PER), :]
            if s < N_HOPS - 1:
                sendbuf[s + 1] = acc
            else:
                out_ref[...] = acc

    return pl.pallas_call(
        body,
        out_shape=jax.ShapeDtypeStruct((M_PER, N), jnp.float32),
        in_specs=[
            pl.BlockSpec(memory_space=pltpu.VMEM),
            pl.BlockSpec(memory_space=pltpu.VMEM),
        ],
        out_specs=pl.BlockSpec(memory_space=pltpu.VMEM),
        scratch_shapes=[
            pltpu.VMEM((M, N), jnp.float32),
            pltpu.VMEM((N_HOPS, M_PER, N), jnp.float32),
            pltpu.VMEM((N_HOPS, M_PER, N), jnp.float32),
            pltpu.SemaphoreType.DMA((N_HOPS,)),
            pltpu.SemaphoreType.DMA((N_HOPS,)),
        ],
        compiler_params=pltpu.CompilerParams(collective_id=0),
    )(A, B)


# device time: 17411 ns/iter; 2.4568x vs baseline; 2.4568x over previous
import jax
import jax.numpy as jnp
from jax import lax
from jax.experimental import pallas as pl
from jax.experimental.pallas import tpu as pltpu

N_DEV = 8
M = 768
N = 768
M_PER = M // N_DEV
N_PEERS = N_DEV - 1


def kernel(A, B):
    def body(a_ref, b_ref, out_ref, part_ref, partbf_ref, recvbuf,
             send_sems, recv_sems):
        p = lax.axis_index("i")

        barrier_sem = pltpu.get_barrier_semaphore()
        for d in range(N_PEERS):
            nbr = jnp.mod(p + 1 + d, N_DEV)
            pl.semaphore_signal(
                barrier_sem, inc=1,
                device_id=(nbr,), device_id_type=pl.DeviceIdType.MESH,
            )
        pl.semaphore_wait(barrier_sem, N_PEERS)

        part_ref[...] = jnp.dot(
            a_ref[...].astype(jnp.bfloat16),
            b_ref[...].astype(jnp.bfloat16),
            preferred_element_type=jnp.float32,
        )
        partbf_ref[...] = part_ref[...].astype(jnp.bfloat16)

        sends = []
        for d in range(N_PEERS):
            j = jnp.mod(p + 1 + d, N_DEV)
            rdma = pltpu.make_async_remote_copy(
                src_ref=partbf_ref.at[pl.ds(j * M_PER, M_PER), :],
                dst_ref=recvbuf.at[N_PEERS - 1 - d],
                send_sem=send_sems.at[d],
                recv_sem=recv_sems.at[N_PEERS - 1 - d],
                device_id=(j,),
                device_id_type=pl.DeviceIdType.MESH,
            )
            rdma.start()
            sends.append(rdma)

        acc = part_ref[pl.ds(p * M_PER, M_PER), :]
        for s in range(N_PEERS):
            recv = pltpu.make_async_remote_copy(
                src_ref=partbf_ref.at[pl.ds(0, M_PER), :],
                dst_ref=recvbuf.at[s],
                send_sem=send_sems.at[0],
                recv_sem=recv_sems.at[s],
                device_id=(p,),
                device_id_type=pl.DeviceIdType.MESH,
            )
            recv.wait_recv()
            acc = acc + recvbuf[s].astype(jnp.float32)
        out_ref[...] = acc

        for rdma in sends:
            rdma.wait_send()

    return pl.pallas_call(
        body,
        out_shape=jax.ShapeDtypeStruct((M_PER, N), jnp.float32),
        in_specs=[
            pl.BlockSpec(memory_space=pltpu.VMEM),
            pl.BlockSpec(memory_space=pltpu.VMEM),
        ],
        out_specs=pl.BlockSpec(memory_space=pltpu.VMEM),
        scratch_shapes=[
            pltpu.VMEM((M, N), jnp.float32),
            pltpu.VMEM((M, N), jnp.bfloat16),
            pltpu.VMEM((N_PEERS, M_PER, N), jnp.bfloat16),
            pltpu.SemaphoreType.DMA((N_PEERS,)),
            pltpu.SemaphoreType.DMA((N_PEERS,)),
        ],
        compiler_params=pltpu.CompilerParams(collective_id=0),
    )(A, B)


# device time: 17133 ns/iter; 2.4966x vs baseline; 1.0162x over previous
import jax
import jax.numpy as jnp
from jax import lax
from jax.experimental import pallas as pl
from jax.experimental.pallas import tpu as pltpu

N_DEV = 8
M = 768
N = 768
M_PER = M // N_DEV
N_PEERS = N_DEV - 1


def kernel(A, B):
    def body(a_ref, b_ref, out_ref, sendbuf, recvbuf, send_sems, recv_sems):
        p = lax.axis_index("i")

        barrier_sem = pltpu.get_barrier_semaphore()
        for d in range(N_PEERS):
            nbr = jnp.mod(p + 1 + d, N_DEV)
            pl.semaphore_signal(
                barrier_sem, inc=1,
                device_id=(nbr,), device_id_type=pl.DeviceIdType.MESH,
            )

        b_bf = b_ref[...].astype(jnp.bfloat16)

        sends = []
        for d in range(N_PEERS):
            j = jnp.mod(p + 1 + d, N_DEV)
            rows = a_ref[pl.ds(j * M_PER, M_PER), :].astype(jnp.bfloat16)
            sendbuf[d] = jnp.dot(
                rows, b_bf, preferred_element_type=jnp.float32
            ).astype(jnp.bfloat16)
            if d == 0:
                pl.semaphore_wait(barrier_sem, N_PEERS)
            rdma = pltpu.make_async_remote_copy(
                src_ref=sendbuf.at[d],
                dst_ref=recvbuf.at[N_PEERS - 1 - d],
                send_sem=send_sems.at[d],
                recv_sem=recv_sems.at[N_PEERS - 1 - d],
                device_id=(j,),
                device_id_type=pl.DeviceIdType.MESH,
            )
            rdma.start()
            sends.append(rdma)

        acc = jnp.dot(
            a_ref[pl.ds(p * M_PER, M_PER), :].astype(jnp.bfloat16), b_bf,
            preferred_element_type=jnp.float32,
        )
        for s in range(N_PEERS):
            recv = pltpu.make_async_remote_copy(
                src_ref=sendbuf.at[0],
                dst_ref=recvbuf.at[s],
                send_sem=send_sems.at[0],
                recv_sem=recv_sems.at[s],
                device_id=(p,),
                device_id_type=pl.DeviceIdType.MESH,
            )
            recv.wait_recv()
            acc = acc + recvbuf[s].astype(jnp.float32)
        out_ref[...] = acc

        for rdma in sends:
            rdma.wait_send()

    return pl.pallas_call(
        body,
        out_shape=jax.ShapeDtypeStruct((M_PER, N), jnp.float32),
        in_specs=[
            pl.BlockSpec(memory_space=pltpu.VMEM),
            pl.BlockSpec(memory_space=pltpu.VMEM),
        ],
        out_specs=pl.BlockSpec(memory_space=pltpu.VMEM),
        scratch_shapes=[
            pltpu.VMEM((N_PEERS, M_PER, N), jnp.bfloat16),
            pltpu.VMEM((N_PEERS, M_PER, N), jnp.bfloat16),
            pltpu.SemaphoreType.DMA((N_PEERS,)),
            pltpu.SemaphoreType.DMA((N_PEERS,)),
        ],
        compiler_params=pltpu.CompilerParams(collective_id=0),
    )(A, B)


# device time: 16672 ns/iter; 2.5657x vs baseline; 1.0277x over previous
import jax
import jax.numpy as jnp
from jax import lax
from jax.experimental import pallas as pl
from jax.experimental.pallas import tpu as pltpu

N_DEV = 8
M = 768
N = 768
M_PER = M // N_DEV
N_PEERS = N_DEV - 1


def kernel(A, B):
    def body(a_ref, b_ref, out_ref, sendbuf, recvbuf, send_sems, recv_sems):
        p = lax.axis_index("i")

        barrier_sem = pltpu.get_barrier_semaphore()
        for d in range(N_PEERS):
            nbr = jnp.mod(p + 1 + d, N_DEV)
            pl.semaphore_signal(
                barrier_sem, inc=1,
                device_id=(nbr,), device_id_type=pl.DeviceIdType.MESH,
            )

        b_bf = b_ref[...].astype(jnp.bfloat16)

        sends = []
        for d in range(N_PEERS):
            j = jnp.mod(p + 1 + d, N_DEV)
            rows = a_ref[pl.ds(j * M_PER, M_PER), :].astype(jnp.bfloat16)
            sendbuf[d] = jnp.dot(
                rows, b_bf, preferred_element_type=jnp.float32
            ).astype(jnp.bfloat16)
            if d == 0:
                pl.semaphore_wait(barrier_sem, N_PEERS)
            rdma = pltpu.make_async_remote_copy(
                src_ref=sendbuf.at[d],
                dst_ref=recvbuf.at[N_PEERS - 1 - d],
                send_sem=send_sems.at[d],
                recv_sem=recv_sems.at[N_PEERS - 1 - d],
                device_id=(j,),
                device_id_type=pl.DeviceIdType.MESH,
            )
            rdma.start()
            sends.append(rdma)

        acc = jnp.dot(
            a_ref[pl.ds(p * M_PER, M_PER), :].astype(jnp.bfloat16), b_bf,
            preferred_element_type=jnp.float32,
        )
        for s in reversed(range(N_PEERS)):
            recv = pltpu.make_async_remote_copy(
                src_ref=sendbuf.at[0],
                dst_ref=recvbuf.at[s],
                send_sem=send_sems.at[0],
                recv_sem=recv_sems.at[s],
                device_id=(p,),
                device_id_type=pl.DeviceIdType.MESH,
            )
            recv.wait_recv()
            acc = acc + recvbuf[s].astype(jnp.float32)
        out_ref[...] = acc

        for rdma in sends:
            rdma.wait_send()

    return pl.pallas_call(
        body,
        out_shape=jax.ShapeDtypeStruct((M_PER, N), jnp.float32),
        in_specs=[
            pl.BlockSpec(memory_space=pltpu.VMEM),
            pl.BlockSpec(memory_space=pltpu.VMEM),
        ],
        out_specs=pl.BlockSpec(memory_space=pltpu.VMEM),
        scratch_shapes=[
            pltpu.VMEM((N_PEERS, M_PER, N), jnp.bfloat16),
            pltpu.VMEM((N_PEERS, M_PER, N), jnp.bfloat16),
            pltpu.SemaphoreType.DMA((N_PEERS,)),
            pltpu.SemaphoreType.DMA((N_PEERS,)),
        ],
        compiler_params=pltpu.CompilerParams(collective_id=0),
    )(A, B)
